# Initial kernel scaffold; baseline (speedup 1.0000x reference)
import jax
import jax.numpy as jnp
from jax.experimental import pallas as pl


def kernel(X, V):
    raise NotImplementedError("write your pallas kernel here")


# speedup vs baseline: 3.3047x; 3.3047x over previous
import functools

import jax
import jax.numpy as jnp
from jax.experimental import pallas as pl

_N = 32768
_D = 256
_K = 1024
_BN = 2048


def _kmeans_block(x_ref, v_ref, o_ref):
    x = x_ref[...]
    v = v_ref[...]
    x2 = jnp.sum(x * x, axis=1, keepdims=True)
    v2 = jnp.sum(v * v, axis=1)[None, :]
    xv = jax.lax.dot_general(
        x, v, (((1,), (1,)), ((), ())),
        preferred_element_type=jnp.float32)
    d2 = x2 + v2 - 2.0 * xv
    dist = jnp.sqrt(jnp.maximum(d2, 1e-12))
    cols = jax.lax.broadcasted_iota(jnp.int32, dist.shape, 1)
    dmin = jnp.min(dist, axis=1, keepdims=True)
    first = jnp.min(jnp.where(dist == dmin, cols, _K), axis=1, keepdims=True)
    o_ref[...] = jnp.where(cols == first, dist, 0.0)


@jax.jit
def kernel(X, V):
    grid = (_N // _BN,)
    return pl.pallas_call(
        _kmeans_block,
        grid=grid,
        in_specs=[
            pl.BlockSpec((_BN, _D), lambda i: (i, 0)),
            pl.BlockSpec((_K, _D), lambda i: (0, 0)),
        ],
        out_specs=pl.BlockSpec((_BN, _K), lambda i: (i, 0)),
        out_shape=jax.ShapeDtypeStruct((_N, _K), jnp.float32),
    )(X, V)


# speedup vs baseline: 4.5054x; 1.3633x over previous
import jax
import jax.numpy as jnp
from jax.experimental import pallas as pl

_N = 32768
_D = 256
_K = 1024
_BN = 2048


def _kmeans_block(x_ref, v_ref, o_ref):
    x = x_ref[...]
    v = v_ref[...]
    x2 = jnp.sum(x * x, axis=1, keepdims=True)
    v2 = jnp.sum(v * v, axis=1)[None, :]
    xv2 = jax.lax.dot_general(
        x * 2.0, v, (((1,), (1,)), ((), ())),
        preferred_element_type=jnp.float32)
    d2 = (x2 + v2) - xv2
    cols = jax.lax.broadcasted_iota(jnp.int32, d2.shape, 1)
    d2min = jnp.min(d2, axis=1, keepdims=True)
    first = jnp.min(jnp.where(d2 == d2min, cols, _K), axis=1, keepdims=True)
    dmin = jnp.sqrt(jnp.maximum(d2min, 1e-12))
    o_ref[...] = jnp.where(cols == first, dmin, 0.0)


@jax.jit
def kernel(X, V):
    grid = (_N // _BN,)
    return pl.pallas_call(
        _kmeans_block,
        grid=grid,
        in_specs=[
            pl.BlockSpec((_BN, _D), lambda i: (i, 0)),
            pl.BlockSpec((_K, _D), lambda i: (0, 0)),
        ],
        out_specs=pl.BlockSpec((_BN, _K), lambda i: (i, 0)),
        out_shape=jax.ShapeDtypeStruct((_N, _K), jnp.float32),
    )(X, V)


# speedup vs baseline: 4.8043x; 1.0663x over previous
import jax
import jax.numpy as jnp
from jax.experimental import pallas as pl
from jax.experimental.pallas import tpu as pltpu

_N = 32768
_D = 256
_K = 1024
_BN = 2048


def _kmeans_block(x_ref, v_ref, o_ref, v2s_ref, v2n_ref):
    @pl.when(pl.program_id(0) == 0)
    def _prep():
        v = v_ref[...]
        v2s_ref[...] = v * 2.0
        v2n_ref[...] = jnp.sum(v * v, axis=1)[None, :]

    x = x_ref[...]
    x2 = jnp.sum(x * x, axis=1, keepdims=True)
    v2 = v2n_ref[...]
    xv2 = jax.lax.dot_general(
        x, v2s_ref[...], (((1,), (1,)), ((), ())),
        preferred_element_type=jnp.float32)
    d2 = (x2 + v2) - xv2
    cols = jax.lax.broadcasted_iota(jnp.int32, d2.shape, 1)
    d2min = jnp.min(d2, axis=1, keepdims=True)
    first = jnp.min(jnp.where(d2 == d2min, cols, _K), axis=1, keepdims=True)
    dmin = jnp.sqrt(jnp.maximum(d2min, 1e-12))
    o_ref[...] = jnp.where(cols == first, dmin, 0.0)


@jax.jit
def kernel(X, V):
    grid = (_N // _BN,)
    return pl.pallas_call(
        _kmeans_block,
        grid=grid,
        in_specs=[
            pl.BlockSpec((_BN, _D), lambda i: (i, 0)),
            pl.BlockSpec((_K, _D), lambda i: (0, 0)),
        ],
        out_specs=pl.BlockSpec((_BN, _K), lambda i: (i, 0)),
        out_shape=jax.ShapeDtypeStruct((_N, _K), jnp.float32),
        scratch_shapes=[
            pltpu.VMEM((_K, _D), jnp.float32),
            pltpu.VMEM((1, _K), jnp.float32),
        ],
    )(X, V)


# speedup vs baseline: 5.5377x; 1.1527x over previous
import jax
import jax.numpy as jnp
from jax.experimental import pallas as pl
from jax.experimental.pallas import tpu as pltpu

_N = 32768
_D = 256
_K = 1024
_BN = 2048
_C = 128


def _kmeans_block(x_ref, v_ref, o_ref, v2s_ref, v2n_ref):
    @pl.when(pl.program_id(0) == 0)
    def _prep():
        v = v_ref[...]
        v2s_ref[...] = v * 2.0
        v2n_ref[...] = jnp.sum(v * v, axis=1)[None, :]

    x = x_ref[...]
    x2 = jnp.sum(x * x, axis=1, keepdims=True)
    v2 = v2n_ref[...]
    xv2 = jax.lax.dot_general(
        x, v2s_ref[...], (((1,), (1,)), ((), ())),
        preferred_element_type=jnp.float32)

    m = (x2 + v2[:, :_C]) - xv2[:, :_C]
    ci = jnp.zeros(m.shape, jnp.int32)
    for i in range(1, _K // _C):
        t = (x2 + v2[:, i * _C:(i + 1) * _C]) - xv2[:, i * _C:(i + 1) * _C]
        lt = t < m
        ci = jnp.where(lt, i, ci)
        m = jnp.minimum(m, t)

    lanes = jax.lax.broadcasted_iota(jnp.int32, m.shape, 1)
    mv = jnp.min(m, axis=1, keepdims=True)
    col = ci * _C + lanes
    first = jnp.min(jnp.where(m == mv, col, _K), axis=1, keepdims=True)
    dmin = jnp.sqrt(jnp.maximum(mv, 1e-12))

    cols = jax.lax.broadcasted_iota(jnp.int32, (_BN, _K), 1)
    o_ref[...] = jnp.where(cols == first, dmin, 0.0)


@jax.jit
def kernel(X, V):
    grid = (_N // _BN,)
    return pl.pallas_call(
        _kmeans_block,
        grid=grid,
        in_specs=[
            pl.BlockSpec((_BN, _D), lambda i: (i, 0)),
            pl.BlockSpec((_K, _D), lambda i: (0, 0)),
        ],
        out_specs=pl.BlockSpec((_BN, _K), lambda i: (i, 0)),
        out_shape=jax.ShapeDtypeStruct((_N, _K), jnp.float32),
        scratch_shapes=[
            pltpu.VMEM((_K, _D), jnp.float32),
            pltpu.VMEM((1, _K), jnp.float32),
        ],
    )(X, V)
